# baseline (speedup 1.0000x reference)
import functools

import jax
import jax.numpy as jnp
from jax import lax
from jax.experimental import pallas as pl
from jax.experimental.pallas import tpu as pltpu
from jax.experimental.pallas import tpu_sc as plsc

N_NODES = 100000
D = 128
B = 4096
T = 5
S1, S2 = 5, 2
HID0, HID1 = 128, 64
NCLS = 16

NC, NS = 2, 16
NW = NC * NS
CH = 128

N1 = T * S1 * B
N2 = T * S2 * S1 * B
NG = B + N1 + N2
NA = B + N1
NOUT = NA + N2 // 2
A_CHUNKS = NA // (NW * CH)
B_CHUNKS = N2 // (NW * CH)
OFF2 = NA


def _sc_gather_body(x_hbm, idx_hbm, out_hbm,
                    iv0, iv1, iv2, rb0, rb1, rb2, sb0, sb1, sb2,
                    si0, si1, si2, sg0, sg1, sg2, sw0, sw1, sw2):
    wid = lax.axis_index("s") * NC + lax.axis_index("c")

    iv = (iv0, iv1, iv2)
    rb = (rb0, rb1, rb2)
    sb = (sb0, sb1, sb2)
    si = (si0, si1, si2)
    sg = (sg0, sg1, sg2)
    sw = (sw0, sw1, sw2)

    def ring(idx_base, out_base, nchunks, summed):
        out_ch = CH // 2 if summed else CH

        def load_idx(i, b):
            pltpu.async_copy(idx_hbm.at[pl.ds(idx_base + i * CH, CH)],
                             iv[b], si[b])

        def gather(b):
            pltpu.async_copy(x_hbm.at[iv[b]], rb[b], sg[b])

        def writeback(i, b):
            src = sb[b] if summed else rb[b]
            pltpu.async_copy(src, out_hbm.at[pl.ds(out_base + i * out_ch,
                                                   out_ch)], sw[b])

        def wait_idx(b):
            pltpu.make_async_copy(idx_hbm.at[pl.ds(0, CH)], iv[b],
                                  si[b]).wait()

        def wait_gather(b):
            pltpu.make_async_copy(x_hbm.at[iv[b]], rb[b], sg[b]).wait()

        def wait_write(b):
            src = sb[b] if summed else rb[b]
            pltpu.make_async_copy(src, out_hbm.at[pl.ds(0, out_ch)],
                                  sw[b]).wait()

        def pair_sum(b):
            def row(k, carry):
                for c in range(D // 16):
                    sl = pl.ds(16 * c, 16)
                    sb[b][k, sl] = rb[b][2 * k, sl] + rb[b][2 * k + 1, sl]
                return carry
            lax.fori_loop(0, CH // 2, row, 0)

        def finish(i, b):
            wait_gather(b)
            if summed:
                pair_sum(b)
            writeback(i, b)

        def step(i, b):
            bp = (b + 2) % 3
            wait_write(b)
            wait_idx(b)
            gather(b)
            finish(i - 1, bp)
            load_idx(i + 2, bp)

        load_idx(0, 0)
        load_idx(1, 1)
        load_idx(2, 2)
        wait_idx(0)
        gather(0)
        wait_idx(1)
        gather(1)
        finish(0, 0)
        load_idx(3, 0)
        wait_idx(2)
        gather(2)
        finish(1, 1)
        load_idx(4, 1)

        def body(i, carry):
            lax.switch(i % 3, [lambda: step(i, 0), lambda: step(i, 1),
                               lambda: step(i, 2)])
            return carry

        lax.fori_loop(3, nchunks, body, 0)

        finish(nchunks - 1, (nchunks - 1) % 3)
        wait_idx(nchunks % 3)
        wait_idx((nchunks + 1) % 3)
        wait_write(0)
        wait_write(1)
        wait_write(2)

    ring(wid * (A_CHUNKS * CH), wid * (A_CHUNKS * CH), A_CHUNKS, False)
    ring(NA + wid * (B_CHUNKS * CH), NA + wid * (B_CHUNKS * CH // 2),
         B_CHUNKS, True)


def _sc_gather(x, idx_all):
    run = functools.partial(
        pl.kernel,
        out_type=jax.ShapeDtypeStruct((NOUT, D), jnp.float32),
        mesh=plsc.VectorSubcoreMesh(core_axis_name="c", subcore_axis_name="s",
                                    num_cores=NC, num_subcores=NS),
        scratch_types=(
            [pltpu.VMEM((CH,), jnp.int32)] * 3
            + [pltpu.VMEM((CH, D), jnp.float32)] * 3
            + [pltpu.VMEM((CH // 2, D), jnp.float32)] * 3
            + [pltpu.SemaphoreType.DMA] * 9
        ),
    )(_sc_gather_body)
    return run(x, idx_all)


def _tc_body(*refs):
    (h0_ref, h1r0, h1r1, h1r2, h1r3, h1r4,
     h2r0, h2r1, h2r2, h2r3, h2r4,
     w0_ref, w1_ref, wp_ref, b0_ref, b1_ref, bp_ref,
     out_ref) = refs
    t = pl.program_id(1)
    f32 = jnp.float32
    h1_refs = (h1r0, h1r1, h1r2, h1r3, h1r4)
    h2_refs = (h2r0, h2r1, h2r2, h2r3, h2r4)

    w0 = w0_ref[...]
    b0 = b0_ref[...]

    h1s = [r[...] for r in h1_refs]
    h1m = (h1s[0] + h1s[1] + h1s[2] + h1s[3] + h1s[4]) / 5.0

    pre = (jnp.dot(jnp.concatenate([h0_ref[...], h1m], axis=1), w0,
                   preferred_element_type=f32) + b0)
    s0_seed = (pre > 1.0).astype(f32)

    acc = None
    for j in range(S1):
        h2m = h2_refs[j][...] * 0.5
        pre_j = (jnp.dot(jnp.concatenate([h1s[j], h2m], axis=1), w0,
                         preferred_element_type=f32) + b0)
        sj = (pre_j > 1.0).astype(f32)
        acc = sj if acc is None else acc + sj
    s0n_mean = acc / 5.0

    pre1 = (jnp.dot(jnp.concatenate([s0_seed, s0n_mean], axis=1), w1_ref[...],
                    preferred_element_type=f32) + b1_ref[...])
    s1 = (pre1 > 1.0).astype(f32)

    contrib = jnp.dot(s1, wp_ref[0], preferred_element_type=f32)

    @pl.when(t == 0)
    def _init():
        out_ref[...] = bp_ref[...] + contrib

    @pl.when(t != 0)
    def _acc():
        out_ref[...] += contrib


def _tc_net(g, w0, w1, wpt, b0, b1, bp2, block_b):
    nb = B // block_b
    grid = (nb, T)
    blk = B // block_b

    def h1_map(j):
        return lambda i, t, j=j: (blk + (t * S1 + j) * blk + i, 0)

    def h2_map(j):
        return lambda i, t, j=j: (OFF2 // block_b + (t * S1 + j) * blk + i, 0)

    slab = pl.BlockSpec((block_b, D), lambda i, t: (i, 0))
    in_specs = (
        [slab]
        + [pl.BlockSpec((block_b, D), h1_map(j)) for j in range(S1)]
        + [pl.BlockSpec((block_b, D), h2_map(j)) for j in range(S1)]
        + [
            pl.BlockSpec((2 * D, HID0), lambda i, t: (0, 0)),
            pl.BlockSpec((2 * HID0, HID1), lambda i, t: (0, 0)),
            pl.BlockSpec((1, HID1, NCLS), lambda i, t: (t, 0, 0)),
            pl.BlockSpec((1, HID0), lambda i, t: (0, 0)),
            pl.BlockSpec((1, HID1), lambda i, t: (0, 0)),
            pl.BlockSpec((1, NCLS), lambda i, t: (0, 0)),
        ]
    )
    args = ([g] * 11) + [w0, w1, wpt, b0, b1, bp2]
    return pl.pallas_call(
        _tc_body,
        grid=grid,
        in_specs=in_specs,
        out_specs=pl.BlockSpec((block_b, NCLS), lambda i, t: (i, 0)),
        out_shape=jax.ShapeDtypeStruct((B, NCLS), jnp.float32),
    )(*args)


def kernel(x, nodes, nbr1, nbr2, W0l, b0l, W0r, b0r, W1l, b1l, W1r, b1r, Wp, bp):
    idx1 = nbr1.reshape(T, B, S1).transpose(0, 2, 1).reshape(-1)
    idx2 = nbr2.reshape(T, B, S1, S2).transpose(0, 2, 1, 3).reshape(-1)
    idx_all = jnp.concatenate(
        [nodes, idx1, idx2, jnp.zeros((2 * CH,), jnp.int32)])

    g = _sc_gather(x, idx_all)

    w0 = jnp.concatenate([W0l, W0r], axis=0)
    w1 = jnp.concatenate([W1l, W1r], axis=0)
    b0 = (b0l + b0r).reshape(1, HID0)
    b1 = (b1l + b1r).reshape(1, HID1)
    bp2 = bp.reshape(1, NCLS)
    wpt = Wp.reshape(T, HID1, NCLS)

    return _tc_net(g, w0, w1, wpt, b0, b1, bp2, block_b=1024)


# speedup vs baseline: 1.8558x; 1.8558x over previous
import functools

import jax
import jax.numpy as jnp
from jax import lax
from jax.experimental import pallas as pl
from jax.experimental.pallas import tpu as pltpu
from jax.experimental.pallas import tpu_sc as plsc

N_NODES = 100000
D = 128
B = 4096
T = 5
S1, S2 = 5, 2
HID0, HID1 = 128, 64
NCLS = 16

NC, NS = 2, 16
NW = NC * NS
CH = 128

N1 = T * S1 * B
N2 = T * S2 * S1 * B
NG = B + N1 + N2
NA = NG
PER_W = NG // NW
NCHUNK = PER_W // CH
OFF1 = B
OFF2 = B + N1


def _sc_gather_body(x_hbm, idx_hbm, out_hbm,
                    iv0, iv1, iv2, rb0, rb1, rb2,
                    si0, si1, si2, sg0, sg1, sg2, sw0, sw1, sw2):
    wid = lax.axis_index("s") * NC + lax.axis_index("c")
    base_w = wid * PER_W

    iv = (iv0, iv1, iv2)
    rb = (rb0, rb1, rb2)
    si = (si0, si1, si2)
    sg = (sg0, sg1, sg2)
    sw = (sw0, sw1, sw2)

    def load_idx(i, b):
        pltpu.async_copy(idx_hbm.at[pl.ds(base_w + i * CH, CH)], iv[b], si[b])

    def gather(b):
        pltpu.async_copy(x_hbm.at[iv[b]], rb[b], sg[b])

    def writeback(i, b):
        pltpu.async_copy(rb[b], out_hbm.at[pl.ds(base_w + i * CH, CH)], sw[b])

    def wait_idx(b):
        pltpu.make_async_copy(idx_hbm.at[pl.ds(0, CH)], iv[b], si[b]).wait()

    def wait_gather(b):
        pltpu.make_async_copy(x_hbm.at[iv[b]], rb[b], sg[b]).wait()

    def wait_write(b):
        pltpu.make_async_copy(rb[b], out_hbm.at[pl.ds(0, CH)], sw[b]).wait()

    def step(i, b):
        bp = (b + 2) % 3
        wait_write(b)
        wait_idx(b)
        gather(b)
        wait_gather(bp)
        writeback(i - 1, bp)
        load_idx(i + 2, bp)

    load_idx(0, 0)
    load_idx(1, 1)
    load_idx(2, 2)
    wait_idx(0)
    gather(0)
    wait_idx(1)
    gather(1)
    wait_gather(0)
    writeback(0, 0)
    load_idx(3, 0)
    wait_idx(2)
    gather(2)
    wait_gather(1)
    writeback(1, 1)
    load_idx(4, 1)

    def body(i, carry):
        lax.switch(i % 3, [lambda: step(i, 0), lambda: step(i, 1),
                           lambda: step(i, 2)])
        return carry

    lax.fori_loop(3, NCHUNK, body, 0)

    wait_gather((NCHUNK - 1) % 3)
    writeback(NCHUNK - 1, (NCHUNK - 1) % 3)
    wait_idx(NCHUNK % 3)
    wait_idx((NCHUNK + 1) % 3)
    wait_write(0)
    wait_write(1)
    wait_write(2)


def _sc_gather(x, idx_all):
    run = functools.partial(
        pl.kernel,
        out_type=jax.ShapeDtypeStruct((NG, D), jnp.float32),
        mesh=plsc.VectorSubcoreMesh(core_axis_name="c", subcore_axis_name="s",
                                    num_cores=NC, num_subcores=NS),
        scratch_types=(
            [pltpu.VMEM((CH,), jnp.int32)] * 3
            + [pltpu.VMEM((CH, D), jnp.float32)] * 3
            + [pltpu.SemaphoreType.DMA] * 9
        ),
    )(_sc_gather_body)
    return run(x, idx_all)


def _tc_body(*refs):
    (h0_ref, h1r0, h1r1, h1r2, h1r3, h1r4,
     h2r0, h2r1, h2r2, h2r3, h2r4, h2r5, h2r6, h2r7, h2r8, h2r9,
     w0_ref, w1_ref, wp_ref, b0_ref, b1_ref, bp_ref,
     out_ref) = refs
    t = pl.program_id(1)
    f32 = jnp.float32
    h1_refs = (h1r0, h1r1, h1r2, h1r3, h1r4)
    h2_refs = (h2r0, h2r1, h2r2, h2r3, h2r4, h2r5, h2r6, h2r7, h2r8, h2r9)

    w0 = w0_ref[...]
    b0 = b0_ref[...]

    h1s = [r[...] for r in h1_refs]
    h1m = (h1s[0] + h1s[1] + h1s[2] + h1s[3] + h1s[4]) / 5.0

    pre = (jnp.dot(jnp.concatenate([h0_ref[...], h1m], axis=1), w0,
                   preferred_element_type=f32) + b0)
    s0_seed = (pre > 1.0).astype(f32)

    acc = None
    for j in range(S1):
        h2m = (h2_refs[j][...] + h2_refs[S1 + j][...]) / 2.0
        pre_j = (jnp.dot(jnp.concatenate([h1s[j], h2m], axis=1), w0,
                         preferred_element_type=f32) + b0)
        sj = (pre_j > 1.0).astype(f32)
        acc = sj if acc is None else acc + sj
    s0n_mean = acc / 5.0

    pre1 = (jnp.dot(jnp.concatenate([s0_seed, s0n_mean], axis=1), w1_ref[...],
                    preferred_element_type=f32) + b1_ref[...])
    s1 = (pre1 > 1.0).astype(f32)

    contrib = jnp.dot(s1, wp_ref[0], preferred_element_type=f32)

    @pl.when(t == 0)
    def _init():
        out_ref[...] = bp_ref[...] + contrib

    @pl.when(t != 0)
    def _acc():
        out_ref[...] += contrib


def _tc_net(g, w0, w1, wpt, b0, b1, bp2, block_b):
    nb = B // block_b
    grid = (nb, T)
    blk = B // block_b

    def h1_map(j):
        return lambda i, t, j=j: (blk + (t * S1 + j) * blk + i, 0)

    def h2_map(q):
        return lambda i, t, q=q: (OFF2 // block_b + (t * S1 * S2 + q) * blk + i, 0)

    slab = pl.BlockSpec((block_b, D), lambda i, t: (i, 0))
    in_specs = (
        [slab]
        + [pl.BlockSpec((block_b, D), h1_map(j)) for j in range(S1)]
        + [pl.BlockSpec((block_b, D), h2_map(q)) for q in range(S1 * S2)]
        + [
            pl.BlockSpec((2 * D, HID0), lambda i, t: (0, 0)),
            pl.BlockSpec((2 * HID0, HID1), lambda i, t: (0, 0)),
            pl.BlockSpec((1, HID1, NCLS), lambda i, t: (t, 0, 0)),
            pl.BlockSpec((1, HID0), lambda i, t: (0, 0)),
            pl.BlockSpec((1, HID1), lambda i, t: (0, 0)),
            pl.BlockSpec((1, NCLS), lambda i, t: (0, 0)),
        ]
    )
    args = ([g] * 16) + [w0, w1, wpt, b0, b1, bp2]
    return pl.pallas_call(
        _tc_body,
        grid=grid,
        in_specs=in_specs,
        out_specs=pl.BlockSpec((block_b, NCLS), lambda i, t: (i, 0)),
        out_shape=jax.ShapeDtypeStruct((B, NCLS), jnp.float32),
    )(*args)


def kernel(x, nodes, nbr1, nbr2, W0l, b0l, W0r, b0r, W1l, b1l, W1r, b1r, Wp, bp):
    idx1 = nbr1.reshape(T, B, S1).transpose(0, 2, 1).reshape(-1)
    idx2 = nbr2.reshape(T, B, S1, S2).transpose(0, 3, 2, 1).reshape(-1)
    idx_all = jnp.concatenate(
        [nodes, idx1, idx2, jnp.zeros((2 * CH,), jnp.int32)])

    g = _sc_gather(x, idx_all)

    w0 = jnp.concatenate([W0l, W0r], axis=0)
    w1 = jnp.concatenate([W1l, W1r], axis=0)
    b0 = (b0l + b0r).reshape(1, HID0)
    b1 = (b1l + b1r).reshape(1, HID1)
    bp2 = bp.reshape(1, NCLS)
    wpt = Wp.reshape(T, HID1, NCLS)

    return _tc_net(g, w0, w1, wpt, b0, b1, bp2, block_b=1024)


# speedup vs baseline: 1.8654x; 1.0051x over previous
import functools

import jax
import jax.numpy as jnp
from jax import lax
from jax.experimental import pallas as pl
from jax.experimental.pallas import tpu as pltpu
from jax.experimental.pallas import tpu_sc as plsc

N_NODES = 100000
D = 128
B = 4096
T = 5
S1, S2 = 5, 2
HID0, HID1 = 128, 64
NCLS = 16

NC, NS = 2, 16
NW = NC * NS
CH = 128

N1 = T * S1 * B
N2 = T * S2 * S1 * B
NG = B + N1 + N2
NA = NG
PER_W = NG // NW
NCHUNK = PER_W // CH
OFF1 = B
OFF2 = B + N1


def _sc_gather_body(x_hbm, idx_hbm, out_hbm,
                    iv0, iv1, iv2, rb0, rb1, rb2,
                    si0, si1, si2, sg0, sg1, sg2, sw0, sw1, sw2):
    wid = lax.axis_index("s") * NC + lax.axis_index("c")
    base_w = wid * PER_W

    iv = (iv0, iv1, iv2)
    rb = (rb0, rb1, rb2)
    si = (si0, si1, si2)
    sg = (sg0, sg1, sg2)
    sw = (sw0, sw1, sw2)

    def load_idx(i, b):
        pltpu.async_copy(idx_hbm.at[pl.ds(base_w + i * CH, CH)], iv[b], si[b])

    def gather(b):
        pltpu.async_copy(x_hbm.at[iv[b]], rb[b], sg[b])

    def writeback(i, b):
        pltpu.async_copy(rb[b], out_hbm.at[pl.ds(base_w + i * CH, CH)], sw[b])

    def wait_idx(b):
        pltpu.make_async_copy(idx_hbm.at[pl.ds(0, CH)], iv[b], si[b]).wait()

    def wait_gather(b):
        pltpu.make_async_copy(x_hbm.at[iv[b]], rb[b], sg[b]).wait()

    def wait_write(b):
        pltpu.make_async_copy(rb[b], out_hbm.at[pl.ds(0, CH)], sw[b]).wait()

    def step(i, b):
        bp = (b + 2) % 3
        wait_write(b)
        wait_idx(b)
        gather(b)
        wait_gather(bp)
        writeback(i - 1, bp)
        load_idx(i + 2, bp)

    load_idx(0, 0)
    load_idx(1, 1)
    load_idx(2, 2)
    wait_idx(0)
    gather(0)
    wait_idx(1)
    gather(1)
    wait_gather(0)
    writeback(0, 0)
    load_idx(3, 0)
    wait_idx(2)
    gather(2)
    wait_gather(1)
    writeback(1, 1)
    load_idx(4, 1)

    def body(i, carry):
        lax.switch(i % 3, [lambda: step(i, 0), lambda: step(i, 1),
                           lambda: step(i, 2)])
        return carry

    lax.fori_loop(3, NCHUNK, body, 0)

    wait_gather((NCHUNK - 1) % 3)
    writeback(NCHUNK - 1, (NCHUNK - 1) % 3)
    wait_idx(NCHUNK % 3)
    wait_idx((NCHUNK + 1) % 3)
    wait_write(0)
    wait_write(1)
    wait_write(2)


def _sc_gather(x, idx_all):
    run = functools.partial(
        pl.kernel,
        out_type=jax.ShapeDtypeStruct((NG, D), jnp.float32),
        mesh=plsc.VectorSubcoreMesh(core_axis_name="c", subcore_axis_name="s",
                                    num_cores=NC, num_subcores=NS),
        scratch_types=(
            [pltpu.VMEM((CH,), jnp.int32)] * 3
            + [pltpu.VMEM((CH, D), jnp.float32)] * 3
            + [pltpu.SemaphoreType.DMA] * 9
        ),
    )(_sc_gather_body)
    return run(x, idx_all)


def _tc_body(*refs):
    (h0_ref, h1r0, h1r1, h1r2, h1r3, h1r4,
     h2r0, h2r1, h2r2, h2r3, h2r4, h2r5, h2r6, h2r7, h2r8, h2r9,
     w0_ref, w1_ref, wp_ref, b0_ref, b1_ref, bp_ref,
     out_ref) = refs
    t = pl.program_id(1)
    f32 = jnp.float32
    h1_refs = (h1r0, h1r1, h1r2, h1r3, h1r4)
    h2_refs = (h2r0, h2r1, h2r2, h2r3, h2r4, h2r5, h2r6, h2r7, h2r8, h2r9)

    w0 = w0_ref[...]
    b0 = b0_ref[...]

    h1s = [r[...] for r in h1_refs]
    h1m = (h1s[0] + h1s[1] + h1s[2] + h1s[3] + h1s[4]) / 5.0

    pre = (jnp.dot(jnp.concatenate([h0_ref[...], h1m], axis=1), w0,
                   preferred_element_type=f32) + b0)
    s0_seed = (pre > 1.0).astype(f32)

    acc = None
    for j in range(S1):
        h2m = (h2_refs[j][...] + h2_refs[S1 + j][...]) / 2.0
        pre_j = (jnp.dot(jnp.concatenate([h1s[j], h2m], axis=1), w0,
                         preferred_element_type=f32) + b0)
        sj = (pre_j > 1.0).astype(f32)
        acc = sj if acc is None else acc + sj
    s0n_mean = acc / 5.0

    pre1 = (jnp.dot(jnp.concatenate([s0_seed, s0n_mean], axis=1), w1_ref[...],
                    preferred_element_type=f32) + b1_ref[...])
    s1 = (pre1 > 1.0).astype(f32)

    contrib = jnp.dot(s1, wp_ref[0], preferred_element_type=f32)

    @pl.when(t == 0)
    def _init():
        out_ref[...] = bp_ref[...] + contrib

    @pl.when(t != 0)
    def _acc():
        out_ref[...] += contrib


def _tc_net(g, w0, w1, wpt, b0, b1, bp2, block_b):
    nb = B // block_b
    grid = (nb, T)
    blk = B // block_b

    def h1_map(j):
        return lambda i, t, j=j: (blk + (t * S1 + j) * blk + i, 0)

    def h2_map(q):
        return lambda i, t, q=q: (OFF2 // block_b + (t * S1 * S2 + q) * blk + i, 0)

    slab = pl.BlockSpec((block_b, D), lambda i, t: (i, 0))
    in_specs = (
        [slab]
        + [pl.BlockSpec((block_b, D), h1_map(j)) for j in range(S1)]
        + [pl.BlockSpec((block_b, D), h2_map(q)) for q in range(S1 * S2)]
        + [
            pl.BlockSpec((2 * D, HID0), lambda i, t: (0, 0)),
            pl.BlockSpec((2 * HID0, HID1), lambda i, t: (0, 0)),
            pl.BlockSpec((1, HID1, NCLS), lambda i, t: (t, 0, 0)),
            pl.BlockSpec((1, HID0), lambda i, t: (0, 0)),
            pl.BlockSpec((1, HID1), lambda i, t: (0, 0)),
            pl.BlockSpec((1, NCLS), lambda i, t: (0, 0)),
        ]
    )
    args = ([g] * 16) + [w0, w1, wpt, b0, b1, bp2]
    return pl.pallas_call(
        _tc_body,
        grid=grid,
        in_specs=in_specs,
        out_specs=pl.BlockSpec((block_b, NCLS), lambda i, t: (i, 0)),
        out_shape=jax.ShapeDtypeStruct((B, NCLS), jnp.float32),
    )(*args)


def kernel(x, nodes, nbr1, nbr2, W0l, b0l, W0r, b0r, W1l, b1l, W1r, b1r, Wp, bp):
    idx1 = nbr1.reshape(T, B, S1).transpose(0, 2, 1).reshape(-1)
    idx2 = nbr2.reshape(T, B, S1, S2).transpose(0, 3, 2, 1).reshape(-1)
    idx_all = jnp.concatenate(
        [nodes, idx1, idx2, jnp.zeros((2 * CH,), jnp.int32)])

    g = _sc_gather(x, idx_all)

    w0 = jnp.concatenate([W0l, W0r], axis=0)
    w1 = jnp.concatenate([W1l, W1r], axis=0)
    b0 = (b0l + b0r).reshape(1, HID0)
    b1 = (b1l + b1r).reshape(1, HID1)
    bp2 = bp.reshape(1, NCLS)
    wpt = Wp.reshape(T, HID1, NCLS)

    return _tc_net(g, w0, w1, wpt, b0, b1, bp2, block_b=2048)
